# baseline (speedup 1.0000x reference)
import functools
import math

import jax
import jax.numpy as jnp
from jax import lax
from jax.experimental import pallas as pl
from jax.experimental.pallas import tpu as pltpu
from jax.experimental.pallas import tpu_sc as plsc

NUM_MIX = 16
_HALF_LOG_2PI = 0.5 * math.log(2.0 * math.pi)
_LN2 = math.log(2.0)


def _vlog(v):
    bits = plsc.bitcast(v, jnp.int32)
    e = jnp.right_shift(bits, 23) - 127
    m_bits = jnp.bitwise_or(jnp.bitwise_and(bits, 0x007FFFFF), 0x3F800000)
    m = plsc.bitcast(m_bits, jnp.float32)
    y = (m - 1.0) / (m + 1.0)
    y2 = y * y
    ln_m = y * (2.0 + y2 * (2.0 / 3.0 + y2 * (2.0 / 5.0 + y2 * (
        2.0 / 7.0 + y2 * (2.0 / 9.0)))))
    return e.astype(jnp.float32) * _LN2 + ln_m


def _vgather(vec, idx):
    return lax.gather(
        vec, idx[:, None],
        dimension_numbers=lax.GatherDimensionNumbers(
            offset_dims=(), collapsed_slice_dims=(0,),
            start_index_map=(0,)),
        slice_sizes=(1,),
        mode=lax.GatherScatterMode.PROMISE_IN_BOUNDS)


@functools.partial(jax.jit, static_argnums=(1, 2))
def _sc_logpdf(args, n, lanes):
    z, x, probs, mu, sd = args
    info = plsc.get_sparse_core_info()
    nw = info.num_cores * info.num_subcores
    per_w = n // nw
    chunk = 8192
    nbuf = 4
    n_chunks = per_w // chunk
    mesh = plsc.VectorSubcoreMesh(core_axis_name="c", subcore_axis_name="s")

    @functools.partial(
        pl.kernel,
        out_type=jax.ShapeDtypeStruct((n,), jnp.float32),
        mesh=mesh,
        compiler_params=pltpu.CompilerParams(needs_layout_passes=False),
        scratch_types=[
            pltpu.VMEM((NUM_MIX,), jnp.float32),
            pltpu.VMEM((NUM_MIX,), jnp.float32),
            pltpu.VMEM((NUM_MIX,), jnp.float32),
            [pltpu.VMEM((chunk,), jnp.int32) for _ in range(4)],
            [pltpu.VMEM((chunk,), jnp.float32) for _ in range(4)],
            [pltpu.VMEM((chunk,), jnp.float32) for _ in range(4)],
            [pltpu.SemaphoreType.DMA for _ in range(4)],
            [pltpu.SemaphoreType.DMA for _ in range(4)],
        ],
    )
    def sc_kernel(z_hbm, x_hbm, p_hbm, mu_hbm, sd_hbm, out_hbm,
                  k_v, mu_v, si_v, z_v, x_v, o_v, in_sem, out_sem):
        wid = lax.axis_index("s") * info.num_cores + lax.axis_index("c")
        base0 = wid * per_w

        def start_in(g):
            b = g % nbuf
            base = base0 + g * chunk
            dz = pltpu.async_copy(z_hbm.at[pl.ds(base, chunk)], z_v[b],
                                  in_sem[b])
            dx = pltpu.async_copy(x_hbm.at[pl.ds(base, chunk)], x_v[b],
                                  in_sem[b])
            return (dz, dx)

        for gp in range(nbuf - 1):
            start_in(gp)

        pltpu.sync_copy(mu_hbm, mu_v)
        pltpu.sync_copy(sd_hbm, si_v)
        pltpu.sync_copy(p_hbm, k_v)
        sd_vec = si_v[...]
        ln_sd = _vlog(sd_vec)
        k_v[...] = _vlog(k_v[...]) - ln_sd - _HALF_LOG_2PI
        si_v[...] = 1.0 / sd_vec

        start_in(nbuf - 1)

        @pl.loop(0, n_chunks, step=nbuf)
        def _(g):
            for b in range(nbuf):
                gg = g + b
                base = base0 + gg * chunk
                pltpu.make_async_copy(z_hbm.at[pl.ds(base, chunk)], z_v[b],
                                      in_sem[b]).wait()
                pltpu.make_async_copy(x_hbm.at[pl.ds(base, chunk)], x_v[b],
                                      in_sem[b]).wait()

                @pl.when(gg >= nbuf)
                def _():
                    pltpu.make_async_copy(
                        o_v[b], out_hbm.at[pl.ds(base0, chunk)],
                        out_sem[b]).wait()

                kvec = k_v[...]
                mvec = mu_v[...]
                svec = si_v[...]

                @plsc.parallel_loop(0, chunk, lanes, unroll=8)
                def _(i):
                    zv = z_v[b][pl.ds(i, lanes)]
                    xv = x_v[b][pl.ds(i, lanes)]
                    kg = _vgather(kvec, zv)
                    mg = _vgather(mvec, zv)
                    sg = _vgather(svec, zv)
                    t = (xv - mg) * sg
                    o_v[b][pl.ds(i, lanes)] = kg - 0.5 * t * t

                pltpu.async_copy(
                    o_v[b], out_hbm.at[pl.ds(base, chunk)], out_sem[b])

                @pl.when(gg + nbuf < n_chunks)
                def _():
                    nxt = base0 + (gg + nbuf) * chunk
                    pltpu.async_copy(z_hbm.at[pl.ds(nxt, chunk)], z_v[b],
                                     in_sem[b])
                    pltpu.async_copy(x_hbm.at[pl.ds(nxt, chunk)], x_v[b],
                                     in_sem[b])

        for b in range(nbuf):
            pltpu.make_async_copy(
                o_v[b], out_hbm.at[pl.ds(base0, chunk)], out_sem[b]).wait()

    return sc_kernel(z, x, probs, mu, sd)


def _sc_kernel_entry(z, x, mixture_probs, means, stds):
    n = z.shape[0]
    info = plsc.get_sparse_core_info()
    return _sc_logpdf(
        (z, x, mixture_probs.astype(jnp.float32),
         means.astype(jnp.float32), stds.astype(jnp.float32)),
        n, info.num_lanes)


def _tc_body(z_ref, x_ref, k_ref, mu_ref, si_ref, o_ref):
    z = z_ref[...]
    x = x_ref[...]
    bshape = (z.shape[0], 128)
    kb = jnp.broadcast_to(k_ref[0:1, :], bshape)
    mb = jnp.broadcast_to(mu_ref[0:1, :], bshape)
    sb = jnp.broadcast_to(si_ref[0:1, :], bshape)
    kg = jnp.take_along_axis(kb, z, axis=1)
    mg = jnp.take_along_axis(mb, z, axis=1)
    sg = jnp.take_along_axis(sb, z, axis=1)
    t = (x - mg) * sg
    o_ref[...] = kg - 0.5 * t * t


def kernel(z, x, mixture_probs, means, stds):
    n = z.shape[0]
    k16 = jnp.log(mixture_probs) - jnp.log(stds) - _HALF_LOG_2PI
    si16 = 1.0 / stds
    k8 = jnp.zeros((8, 128), jnp.float32).at[0, :16].set(k16)
    mu8 = jnp.zeros((8, 128), jnp.float32).at[0, :16].set(means)
    si8 = jnp.zeros((8, 128), jnp.float32).at[0, :16].set(si16)
    B = 512
    f = pl.pallas_call(
        _tc_body,
        grid=(n // (B * 512),),
        in_specs=[
            pl.BlockSpec((B, 512), lambda i: (i, 0)),
            pl.BlockSpec((B, 512), lambda i: (i, 0)),
            pl.BlockSpec((8, 128), lambda i: (0, 0)),
            pl.BlockSpec((8, 128), lambda i: (0, 0)),
            pl.BlockSpec((8, 128), lambda i: (0, 0)),
        ],
        out_specs=pl.BlockSpec((B, 512), lambda i: (i, 0)),
        out_shape=jax.ShapeDtypeStruct((n // 512, 512), jnp.float32),
    )
    return f(z.reshape(n // 512, 512), x.reshape(n // 512, 512),
             k8, mu8, si8).reshape(n)


# speedup vs baseline: 2.6460x; 2.6460x over previous
import functools
import math

import jax
import jax.numpy as jnp
from jax import lax
from jax.experimental import pallas as pl
from jax.experimental.pallas import tpu as pltpu
from jax.experimental.pallas import tpu_sc as plsc

NUM_MIX = 16
_HALF_LOG_2PI = 0.5 * math.log(2.0 * math.pi)
_LN2 = math.log(2.0)


def _vlog(v):
    bits = plsc.bitcast(v, jnp.int32)
    e = jnp.right_shift(bits, 23) - 127
    m_bits = jnp.bitwise_or(jnp.bitwise_and(bits, 0x007FFFFF), 0x3F800000)
    m = plsc.bitcast(m_bits, jnp.float32)
    y = (m - 1.0) / (m + 1.0)
    y2 = y * y
    ln_m = y * (2.0 + y2 * (2.0 / 3.0 + y2 * (2.0 / 5.0 + y2 * (
        2.0 / 7.0 + y2 * (2.0 / 9.0)))))
    return e.astype(jnp.float32) * _LN2 + ln_m


def _vgather(vec, idx):
    return lax.gather(
        vec, idx[:, None],
        dimension_numbers=lax.GatherDimensionNumbers(
            offset_dims=(), collapsed_slice_dims=(0,),
            start_index_map=(0,)),
        slice_sizes=(1,),
        mode=lax.GatherScatterMode.PROMISE_IN_BOUNDS)


@functools.partial(jax.jit, static_argnums=(1, 2))
def _sc_logpdf(args, n, lanes):
    z, x, probs, mu, sd = args
    info = plsc.get_sparse_core_info()
    nw = info.num_cores * info.num_subcores
    per_w = n // nw
    chunk = 8192
    nbuf = 4
    n_chunks = per_w // chunk
    mesh = plsc.VectorSubcoreMesh(core_axis_name="c", subcore_axis_name="s")

    @functools.partial(
        pl.kernel,
        out_type=jax.ShapeDtypeStruct((n,), jnp.float32),
        mesh=mesh,
        compiler_params=pltpu.CompilerParams(needs_layout_passes=False),
        scratch_types=[
            pltpu.VMEM((NUM_MIX,), jnp.float32),
            pltpu.VMEM((NUM_MIX,), jnp.float32),
            pltpu.VMEM((NUM_MIX,), jnp.float32),
            [pltpu.VMEM((chunk,), jnp.int32) for _ in range(4)],
            [pltpu.VMEM((chunk,), jnp.float32) for _ in range(4)],
            [pltpu.VMEM((chunk,), jnp.float32) for _ in range(4)],
            [pltpu.SemaphoreType.DMA for _ in range(4)],
            [pltpu.SemaphoreType.DMA for _ in range(4)],
        ],
    )
    def sc_kernel(z_hbm, x_hbm, p_hbm, mu_hbm, sd_hbm, out_hbm,
                  k_v, mu_v, si_v, z_v, x_v, o_v, in_sem, out_sem):
        wid = lax.axis_index("s") * info.num_cores + lax.axis_index("c")
        base0 = wid * per_w

        def start_in(g):
            b = g % nbuf
            base = base0 + g * chunk
            dz = pltpu.async_copy(z_hbm.at[pl.ds(base, chunk)], z_v[b],
                                  in_sem[b])
            dx = pltpu.async_copy(x_hbm.at[pl.ds(base, chunk)], x_v[b],
                                  in_sem[b])
            return (dz, dx)

        for gp in range(nbuf - 1):
            start_in(gp)

        pltpu.sync_copy(mu_hbm, mu_v)
        pltpu.sync_copy(sd_hbm, si_v)
        pltpu.sync_copy(p_hbm, k_v)
        sd_vec = si_v[...]
        ln_sd = _vlog(sd_vec)
        k_v[...] = _vlog(k_v[...]) - ln_sd - _HALF_LOG_2PI
        si_v[...] = 1.0 / sd_vec

        start_in(nbuf - 1)

        @pl.loop(0, n_chunks, step=nbuf)
        def _(g):
            for b in range(nbuf):
                gg = g + b
                base = base0 + gg * chunk
                pltpu.make_async_copy(z_hbm.at[pl.ds(base, chunk)], z_v[b],
                                      in_sem[b]).wait()
                pltpu.make_async_copy(x_hbm.at[pl.ds(base, chunk)], x_v[b],
                                      in_sem[b]).wait()

                @pl.when(gg >= nbuf)
                def _():
                    pltpu.make_async_copy(
                        o_v[b], out_hbm.at[pl.ds(base0, chunk)],
                        out_sem[b]).wait()

                kvec = k_v[...]
                mvec = mu_v[...]
                svec = si_v[...]

                @plsc.parallel_loop(0, chunk, lanes, unroll=8)
                def _(i):
                    zv = z_v[b][pl.ds(i, lanes)]
                    xv = x_v[b][pl.ds(i, lanes)]
                    kg = _vgather(kvec, zv)
                    mg = _vgather(mvec, zv)
                    sg = _vgather(svec, zv)
                    t = (xv - mg) * sg
                    o_v[b][pl.ds(i, lanes)] = kg - 0.5 * t * t

                pltpu.async_copy(
                    o_v[b], out_hbm.at[pl.ds(base, chunk)], out_sem[b])

                @pl.when(gg + nbuf < n_chunks)
                def _():
                    nxt = base0 + (gg + nbuf) * chunk
                    pltpu.async_copy(z_hbm.at[pl.ds(nxt, chunk)], z_v[b],
                                     in_sem[b])
                    pltpu.async_copy(x_hbm.at[pl.ds(nxt, chunk)], x_v[b],
                                     in_sem[b])

        for b in range(nbuf):
            pltpu.make_async_copy(
                o_v[b], out_hbm.at[pl.ds(base0, chunk)], out_sem[b]).wait()

    return sc_kernel(z, x, probs, mu, sd)


def kernel(z, x, mixture_probs, means, stds):
    n = z.shape[0]
    info = plsc.get_sparse_core_info()
    return _sc_logpdf(
        (z, x, mixture_probs.astype(jnp.float32),
         means.astype(jnp.float32), stds.astype(jnp.float32)),
        n, info.num_lanes)
